# Initial kernel scaffold; baseline (speedup 1.0000x reference)
import jax
import jax.numpy as jnp
from jax.experimental import pallas as pl


def kernel(query, knowledge_embed, knowledge_full):
    raise NotImplementedError("write your pallas kernel here")


# speedup vs baseline: 1.3771x; 1.3771x over previous
import functools

import jax
import jax.numpy as jnp
from jax import lax
from jax.experimental import pallas as pl
from jax.experimental.pallas import tpu as pltpu

TOPK = 16
EPS = 1e-8
NEG = -1e30

Q = 1024
D = 128
K = 100000
KBLK = 2048
NBLK = (K + KBLK - 1) // KBLK
KPAD = NBLK * KBLK


def _topk_body(query_ref, ke_ref, out_i_ref, out_v_ref, qn_ref, run_v_ref, run_i_ref):
    j = pl.program_id(0)

    @pl.when(j == 0)
    def _init():
        q = query_ref[...]
        qn = q / jnp.clip(jnp.sqrt(jnp.sum(q * q, axis=1, keepdims=True)), EPS, None)
        qn_ref[...] = qn
        run_v_ref[...] = jnp.full((Q, TOPK), NEG, jnp.float32)
        run_i_ref[...] = jnp.zeros((Q, TOPK), jnp.int32)

    ke = ke_ref[...]
    kn = ke / jnp.clip(jnp.sqrt(jnp.sum(ke * ke, axis=1, keepdims=True)), EPS, None)
    sims = lax.dot_general(
        qn_ref[...], kn, (((1,), (1,)), ((), ())),
        preferred_element_type=jnp.float32,
    )

    col = lax.broadcasted_iota(jnp.int32, (Q, KBLK), 1)
    gidx = j * KBLK + col
    sims = jnp.where(gidx < K, sims, NEG)

    bv = []
    bi = []
    for _ in range(TOPK):
        m = jnp.max(sims, axis=1, keepdims=True)
        a = jnp.min(jnp.where(sims == m, col, KBLK), axis=1, keepdims=True)
        bv.append(m)
        bi.append(j * KBLK + a)
        sims = jnp.where(col == a, NEG, sims)

    blk_v = jnp.concatenate(bv, axis=1)
    blk_i = jnp.concatenate(bi, axis=1)

    cv = jnp.concatenate([run_v_ref[...], blk_v], axis=1)
    ci = jnp.concatenate([run_i_ref[...], blk_i], axis=1)
    pos = lax.broadcasted_iota(jnp.int32, (Q, 2 * TOPK), 1)
    nv = []
    ni = []
    for _ in range(TOPK):
        m = jnp.max(cv, axis=1, keepdims=True)
        a = jnp.min(jnp.where(cv == m, pos, 2 * TOPK), axis=1, keepdims=True)
        nv.append(m)
        ni.append(jnp.sum(jnp.where(pos == a, ci, 0), axis=1, keepdims=True))
        cv = jnp.where(pos == a, NEG, cv)

    run_v_ref[...] = jnp.concatenate(nv, axis=1)
    run_i_ref[...] = jnp.concatenate(ni, axis=1)

    @pl.when(j == NBLK - 1)
    def _done():
        out_i_ref[...] = run_i_ref[...]
        out_v_ref[...] = run_v_ref[...]


def _topk_indices(query, knowledge_embed):
    ke_pad = jnp.pad(knowledge_embed, ((0, KPAD - K), (0, 0)))
    out_i, _ = pl.pallas_call(
        _topk_body,
        grid=(NBLK,),
        in_specs=[
            pl.BlockSpec((Q, D), lambda j: (0, 0)),
            pl.BlockSpec((KBLK, D), lambda j: (j, 0)),
        ],
        out_specs=[
            pl.BlockSpec((Q, TOPK), lambda j: (0, 0)),
            pl.BlockSpec((Q, TOPK), lambda j: (0, 0)),
        ],
        out_shape=[
            jax.ShapeDtypeStruct((Q, TOPK), jnp.int32),
            jax.ShapeDtypeStruct((Q, TOPK), jnp.float32),
        ],
        scratch_shapes=[
            pltpu.VMEM((Q, D), jnp.float32),
            pltpu.VMEM((Q, TOPK), jnp.float32),
            pltpu.VMEM((Q, TOPK), jnp.int32),
        ],
    )(query, ke_pad)
    return out_i


def kernel(query, knowledge_embed, knowledge_full):
    indices = _topk_indices(query, knowledge_embed)
    topk_embed = jnp.take(knowledge_embed, indices, axis=0)
    topk_knowledge = jnp.take(knowledge_full, indices, axis=0)
    return (topk_knowledge, topk_embed)


# speedup vs baseline: 1.3845x; 1.0054x over previous
import functools

import jax
import jax.numpy as jnp
from jax import lax
from jax.experimental import pallas as pl
from jax.experimental.pallas import tpu as pltpu
from jax.experimental.pallas import tpu_sc as plsc

TOPK = 16
EPS = 1e-8
NEG = -1e30

Q = 1024
D = 128
K = 100000
KBLK = 2048
NBLK = (K + KBLK - 1) // KBLK
KPAD = NBLK * KBLK


def _topk_body(query_ref, ke_ref, out_i_ref, out_v_ref, qn_ref, run_v_ref, run_i_ref):
    j = pl.program_id(0)

    @pl.when(j == 0)
    def _init():
        q = query_ref[...]
        qn = q / jnp.clip(jnp.sqrt(jnp.sum(q * q, axis=1, keepdims=True)), EPS, None)
        qn_ref[...] = qn
        run_v_ref[...] = jnp.full((Q, TOPK), NEG, jnp.float32)
        run_i_ref[...] = jnp.zeros((Q, TOPK), jnp.int32)

    ke = ke_ref[...]
    kn = ke / jnp.clip(jnp.sqrt(jnp.sum(ke * ke, axis=1, keepdims=True)), EPS, None)
    sims = lax.dot_general(
        qn_ref[...], kn, (((1,), (1,)), ((), ())),
        preferred_element_type=jnp.float32,
    )

    col = lax.broadcasted_iota(jnp.int32, (Q, KBLK), 1)
    gidx = j * KBLK + col
    sims = jnp.where(gidx < K, sims, NEG)

    bv = []
    bi = []
    for _ in range(TOPK):
        m = jnp.max(sims, axis=1, keepdims=True)
        a = jnp.min(jnp.where(sims == m, col, KBLK), axis=1, keepdims=True)
        bv.append(m)
        bi.append(j * KBLK + a)
        sims = jnp.where(col == a, NEG, sims)

    blk_v = jnp.concatenate(bv, axis=1)
    blk_i = jnp.concatenate(bi, axis=1)

    cv = jnp.concatenate([run_v_ref[...], blk_v], axis=1)
    ci = jnp.concatenate([run_i_ref[...], blk_i], axis=1)
    pos = lax.broadcasted_iota(jnp.int32, (Q, 2 * TOPK), 1)
    nv = []
    ni = []
    for _ in range(TOPK):
        m = jnp.max(cv, axis=1, keepdims=True)
        a = jnp.min(jnp.where(cv == m, pos, 2 * TOPK), axis=1, keepdims=True)
        nv.append(m)
        ni.append(jnp.sum(jnp.where(pos == a, ci, 0), axis=1, keepdims=True))
        cv = jnp.where(pos == a, NEG, cv)

    run_v_ref[...] = jnp.concatenate(nv, axis=1)
    run_i_ref[...] = jnp.concatenate(ni, axis=1)

    @pl.when(j == NBLK - 1)
    def _done():
        out_i_ref[...] = run_i_ref[...]
        out_v_ref[...] = run_v_ref[...]


def _topk_indices(query, knowledge_embed):
    ke_pad = jnp.pad(knowledge_embed, ((0, KPAD - K), (0, 0)))
    out_i, _ = pl.pallas_call(
        _topk_body,
        grid=(NBLK,),
        in_specs=[
            pl.BlockSpec((Q, D), lambda j: (0, 0)),
            pl.BlockSpec((KBLK, D), lambda j: (j, 0)),
        ],
        out_specs=[
            pl.BlockSpec((Q, TOPK), lambda j: (0, 0)),
            pl.BlockSpec((Q, TOPK), lambda j: (0, 0)),
        ],
        out_shape=[
            jax.ShapeDtypeStruct((Q, TOPK), jnp.int32),
            jax.ShapeDtypeStruct((Q, TOPK), jnp.float32),
        ],
        scratch_shapes=[
            pltpu.VMEM((Q, D), jnp.float32),
            pltpu.VMEM((Q, TOPK), jnp.float32),
            pltpu.VMEM((Q, TOPK), jnp.int32),
        ],
    )(query, ke_pad)
    return out_i


def _make_sc_gather(batch, dim):
    info = plsc.get_sparse_core_info()
    nw = info.num_cores * info.num_subcores
    assert batch % (8 * nw) == 0
    b_per_w = batch // nw
    mesh = plsc.VectorSubcoreMesh(core_axis_name="c", subcore_axis_name="s")

    @functools.partial(
        pl.kernel,
        mesh=mesh,
        out_type=jax.ShapeDtypeStruct((batch, dim), jnp.float32),
        scratch_types=[
            pltpu.VMEM((b_per_w,), jnp.int32),
            pltpu.VMEM((b_per_w, dim), jnp.float32),
            pltpu.SemaphoreType.DMA,
        ],
    )
    def gather(table_hbm, idx_hbm, out_hbm, idx_v, rows_v, sem):
        wid = lax.axis_index("s") * info.num_cores + lax.axis_index("c")
        base = wid * b_per_w
        pltpu.sync_copy(idx_hbm.at[pl.ds(base, b_per_w)], idx_v)
        pltpu.async_copy(table_hbm.at[idx_v], rows_v, sem).wait()
        pltpu.sync_copy(rows_v, out_hbm.at[pl.ds(base, b_per_w)])

    return gather


def kernel(query, knowledge_embed, knowledge_full):
    indices = _topk_indices(query, knowledge_embed)
    flat_idx = indices.reshape(Q * TOPK)
    rows = _make_sc_gather(Q * TOPK, D)(knowledge_embed, flat_idx)
    topk_embed = rows.reshape(Q, TOPK, D)
    topk_knowledge = jnp.take(knowledge_full, indices, axis=0)
    return (topk_knowledge, topk_embed)


# speedup vs baseline: 2.5808x; 1.8641x over previous
import functools

import jax
import jax.numpy as jnp
from jax import lax
from jax.experimental import pallas as pl
from jax.experimental.pallas import tpu as pltpu
from jax.experimental.pallas import tpu_sc as plsc

TOPK = 16
EPS = 1e-8
NEG = -1e30

Q = 1024
D = 128
K = 100000
KBLK = 2048
NBLK = (K + KBLK - 1) // KBLK
KPAD = NBLK * KBLK


NSTR = KBLK // 128
BIGI = 2 ** 30


def _topk_body(qn_ref, kn_ref, out_i_ref, run_v_ref, run_i_ref):
    j = pl.program_id(0)

    @pl.when(j == 0)
    def _init():
        run_v_ref[...] = jnp.full((Q, TOPK), NEG, jnp.float32)
        run_i_ref[...] = jnp.zeros((Q, TOPK), jnp.int32)

    sims = lax.dot_general(
        qn_ref[...], kn_ref[...], (((1,), (1,)), ((), ())),
        preferred_element_type=jnp.float32,
    )

    col = lax.broadcasted_iota(jnp.int32, (Q, KBLK), 1)
    sims = jnp.where(j * KBLK + col < K, sims, NEG)

    stripes = [sims[:, s * 128:(s + 1) * 128] for s in range(NSTR)]
    gm = stripes[0]
    for s in range(1, NSTR):
        gm = jnp.maximum(gm, stripes[s])

    lane = lax.broadcasted_iota(jnp.int32, (Q, 128), 1)
    sel = []
    for _ in range(TOPK):
        m = jnp.max(gm, axis=1, keepdims=True)
        a = jnp.min(jnp.where(gm == m, lane, 128), axis=1, keepdims=True)
        sel.append(a)
        gm = jnp.where(lane == a, NEG, gm)
    sel_lanes = jnp.concatenate(sel, axis=1)

    cand_v = jnp.concatenate(
        [jnp.take_along_axis(stripes[s], sel_lanes, axis=1) for s in range(NSTR)],
        axis=1)
    cand_i = jnp.concatenate(
        [j * KBLK + s * 128 + sel_lanes for s in range(NSTR)],
        axis=1)

    cv = jnp.concatenate([run_v_ref[...], cand_v], axis=1)
    ci = jnp.concatenate([run_i_ref[...], cand_i], axis=1)
    nv = []
    ni = []
    for _ in range(TOPK):
        m = jnp.max(cv, axis=1, keepdims=True)
        aid = jnp.min(jnp.where(cv == m, ci, BIGI), axis=1, keepdims=True)
        nv.append(m)
        ni.append(aid)
        cv = jnp.where(ci == aid, NEG, cv)

    run_v_ref[...] = jnp.concatenate(nv, axis=1)
    run_i_ref[...] = jnp.concatenate(ni, axis=1)

    @pl.when(j == NBLK - 1)
    def _done():
        out_i_ref[...] = run_i_ref[...]


def _topk_indices(query, knowledge_embed):
    qn = query / jnp.clip(jnp.linalg.norm(query, axis=-1, keepdims=True), EPS, None)
    kn = knowledge_embed / jnp.clip(
        jnp.linalg.norm(knowledge_embed, axis=-1, keepdims=True), EPS, None)
    kn_pad = jnp.pad(kn, ((0, KPAD - K), (0, 0)))
    out_i = pl.pallas_call(
        _topk_body,
        grid=(NBLK,),
        in_specs=[
            pl.BlockSpec((Q, D), lambda j: (0, 0)),
            pl.BlockSpec((KBLK, D), lambda j: (j, 0)),
        ],
        out_specs=pl.BlockSpec((Q, TOPK), lambda j: (0, 0)),
        out_shape=jax.ShapeDtypeStruct((Q, TOPK), jnp.int32),
        scratch_shapes=[
            pltpu.VMEM((Q, TOPK), jnp.float32),
            pltpu.VMEM((Q, TOPK), jnp.int32),
        ],
    )(qn, kn_pad)
    return out_i


def _make_sc_gather(batch, dim):
    info = plsc.get_sparse_core_info()
    nw = info.num_cores * info.num_subcores
    assert batch % (8 * nw) == 0
    b_per_w = batch // nw
    mesh = plsc.VectorSubcoreMesh(core_axis_name="c", subcore_axis_name="s")

    @functools.partial(
        pl.kernel,
        mesh=mesh,
        out_type=jax.ShapeDtypeStruct((batch, dim), jnp.float32),
        scratch_types=[
            pltpu.VMEM((b_per_w,), jnp.int32),
            pltpu.VMEM((b_per_w, dim), jnp.float32),
            pltpu.SemaphoreType.DMA,
        ],
    )
    def gather(table_hbm, idx_hbm, out_hbm, idx_v, rows_v, sem):
        wid = lax.axis_index("s") * info.num_cores + lax.axis_index("c")
        base = wid * b_per_w
        pltpu.sync_copy(idx_hbm.at[pl.ds(base, b_per_w)], idx_v)
        pltpu.async_copy(table_hbm.at[idx_v], rows_v, sem).wait()
        pltpu.sync_copy(rows_v, out_hbm.at[pl.ds(base, b_per_w)])

    return gather


def kernel(query, knowledge_embed, knowledge_full):
    indices = _topk_indices(query, knowledge_embed)
    flat_idx = indices.reshape(Q * TOPK)
    rows = _make_sc_gather(Q * TOPK, D)(knowledge_embed, flat_idx)
    topk_embed = rows.reshape(Q, TOPK, D)
    topk_knowledge = jnp.take(knowledge_full, indices, axis=0)
    return (topk_knowledge, topk_embed)
